# Initial kernel scaffold; baseline (speedup 1.0000x reference)
import jax
import jax.numpy as jnp
from jax.experimental import pallas as pl


def kernel(dom_a_idx, dom_b_idx, table_a, table_b):
    raise NotImplementedError("write your pallas kernel here")


# speedup vs baseline: 2.3782x; 2.3782x over previous
import functools

import jax
import jax.numpy as jnp
from jax import lax
from jax.experimental import pallas as pl
from jax.experimental.pallas import tpu as pltpu
from jax.experimental.pallas import tpu_sc as plsc

_DIM_A = 32
_DIM_B = 64
_BATCH, _HIST = 16384, 50
_N = _BATCH * _HIST
_NC, _NS = 2, 16
_NW = _NC * _NS
_PER_W = _N // _NW
_CHUNK = 512
_NCH = _PER_W // _CHUNK
_NBUF = 2

_mesh = plsc.VectorSubcoreMesh(core_axis_name="c", subcore_axis_name="s")


@functools.partial(
    pl.kernel,
    mesh=_mesh,
    out_type=[
        jax.ShapeDtypeStruct((_N, _DIM_A), jnp.float32),
        jax.ShapeDtypeStruct((_N, _DIM_B), jnp.float32),
    ],
    scratch_types=[
        pltpu.VMEM((_PER_W,), jnp.int32),
        pltpu.VMEM((_NBUF, _CHUNK, _DIM_A), jnp.float32),
        pltpu.VMEM((_NBUF, _CHUNK, _DIM_B), jnp.float32),
        pltpu.SemaphoreType.DMA,
    ],
    compiler_params=pltpu.CompilerParams(use_tc_tiling_on_sc=False),
)
def _gather_kernel(idx_a_hbm, idx_b_hbm, table_a_hbm, table_b_hbm,
                   out_a_hbm, out_b_hbm, idx_v, rows_a, rows_b, sem):
    wid = lax.axis_index("s") * _NC + lax.axis_index("c")
    base = wid * _PER_W

    def phase(idx_hbm, table_hbm, rows, out_hbm):
        pltpu.sync_copy(idx_hbm.at[pl.ds(base, _PER_W)], idx_v)

        def start(i, slot):
            pltpu.make_async_copy(
                table_hbm.at[idx_v.at[pl.ds(i * _CHUNK, _CHUNK)]],
                rows.at[slot], sem).start()

        def wait(slot):
            pltpu.make_async_copy(
                table_hbm.at[idx_v.at[pl.ds(0, _CHUNK)]],
                rows.at[slot], sem).wait()

        def drain(i, slot):
            wait(slot)
            pltpu.sync_copy(rows.at[slot],
                            out_hbm.at[pl.ds(base + i * _CHUNK, _CHUNK)])

        start(0, 0)
        start(1, 1)

        def body(j):
            for b in range(_NBUF):
                i = j + b
                drain(i, b)
                start(i + _NBUF, b)

        lax.fori_loop(0, (_NCH - _NBUF) // _NBUF,
                      lambda t, _: (body(t * _NBUF), 0)[1], 0)
        for b in range(_NBUF):
            drain(_NCH - _NBUF + b, b)

    phase(idx_a_hbm, table_a_hbm, rows_a, out_a_hbm)
    phase(idx_b_hbm, table_b_hbm, rows_b, out_b_hbm)


def kernel(dom_a_idx, dom_b_idx, table_a, table_b):
    idx_a = dom_a_idx.reshape(_N)
    idx_b = dom_b_idx.reshape(_N)
    out_a, out_b = _gather_kernel(idx_a, idx_b, table_a, table_b)
    return (out_a.reshape(_BATCH, _HIST, _DIM_A),
            out_b.reshape(_BATCH, _HIST, _DIM_B))
